# baseline (speedup 1.0000x reference)
import functools
import math

import jax
import jax.numpy as jnp
from jax import lax
from jax.experimental import pallas as pl
from jax.experimental.pallas import tpu as pltpu
from jax.experimental.pallas import tpu_sc as plsc

N_ENT = 5000
D = 256
E_TRI = 150000

NC = 2
NS = 16
NW = NC * NS
L = 16

C_SCORE = 64
E_PAD = ((E_TRI + NW * C_SCORE - 1) // (NW * C_SCORE)) * (NW * C_SCORE)
PER_W = E_PAD // NW
N_CHUNKS = PER_W // C_SCORE

ROWS_B = 25
N_BLK = N_ENT // ROWS_B
BLK_W = ROWS_B * N_ENT
BLK_WP = ((BLK_W + L - 1) // L) * L
C_SCAT = 1024
UNROLL = 1
N_CHUNKS2 = E_PAD // C_SCAT
N_PASS = (N_BLK + NW - 1) // NW

NEG = -1000000000.0
BIG = 1 << 30


def _mesh():
    return plsc.VectorSubcoreMesh(
        core_axis_name="c", subcore_axis_name="s", num_cores=NC, num_subcores=NS
    )


def _score_kernel(ent_hbm, rel_hbm, h_hbm, t_hbm, r_hbm, out_hbm,
                  hi_v, ti_v, ri_v, hrow_v, trow_v, rrow_v, sc_v, sem):
    wid = lax.axis_index("s") * NC + lax.axis_index("c")
    base_w = wid * PER_W
    lane = lax.iota(jnp.int32, L)

    def chunk(ci, carry):
        base = base_w + ci * C_SCORE
        pltpu.sync_copy(h_hbm.at[pl.ds(base, C_SCORE)], hi_v)
        pltpu.sync_copy(t_hbm.at[pl.ds(base, C_SCORE)], ti_v)
        pltpu.sync_copy(r_hbm.at[pl.ds(base, C_SCORE)], ri_v)
        pltpu.async_copy(ent_hbm.at[hi_v], hrow_v, sem).wait()
        pltpu.async_copy(ent_hbm.at[ti_v], trow_v, sem).wait()
        pltpu.async_copy(rel_hbm.at[ri_v], rrow_v, sem).wait()

        def grp(g, carry2):
            sqv = jnp.zeros((L,), jnp.float32)
            for i in range(L):
                acc = jnp.zeros((L,), jnp.float32)
                row = g * L + i
                for j in range(D // L):
                    hv = hrow_v[row, pl.ds(j * L, L)]
                    rv = rrow_v[row, pl.ds(j * L, L)]
                    tv = trow_v[row, pl.ds(j * L, L)]
                    d = hv + rv - tv
                    acc = acc + d * d
                for k in (8, 4, 2, 1):
                    acc = acc + acc.at[lane ^ k].get(mode="promise_in_bounds")
                sqv = jnp.where(lane == i, acc, sqv)
            sc_v[pl.ds(g * L, L)] = sqv
            return carry2

        lax.fori_loop(0, C_SCORE // L, grp, 0)
        pltpu.sync_copy(sc_v, out_hbm.at[pl.ds(base, C_SCORE)])
        return carry

    lax.fori_loop(0, N_CHUNKS, chunk, 0)


def _make_score_call():
    return functools.partial(
        pl.kernel,
        out_type=jax.ShapeDtypeStruct((E_PAD,), jnp.float32),
        mesh=_mesh(),
        scratch_types=[
            pltpu.VMEM((C_SCORE,), jnp.int32),
            pltpu.VMEM((C_SCORE,), jnp.int32),
            pltpu.VMEM((C_SCORE,), jnp.int32),
            pltpu.VMEM((C_SCORE, D), jnp.float32),
            pltpu.VMEM((C_SCORE, D), jnp.float32),
            pltpu.VMEM((C_SCORE, D), jnp.float32),
            pltpu.VMEM((C_SCORE,), jnp.float32),
            pltpu.SemaphoreType.DMA,
        ],
        compiler_params=pltpu.CompilerParams(needs_layout_passes=False),
    )(_score_kernel)


def _scatter_kernel(h_hbm, t_hbm, s_hbm, out_hbm, blk_v, h_v, t_v, s_v):
    wid = lax.axis_index("s") * NC + lax.axis_index("c")

    for p in range(N_PASS):
        blk = wid + NW * p

        @pl.when(blk < N_BLK)
        def _():
            base_row = blk * ROWS_B

            def zero(i, carry):
                blk_v[pl.ds(i * L, L)] = jnp.zeros((L,), jnp.float32)
                return carry

            lax.fori_loop(0, BLK_WP // L, zero, 0)

            def chunk(ci, carry):
                off = ci * C_SCAT
                pltpu.sync_copy(h_hbm.at[pl.ds(off, C_SCAT)], h_v)
                pltpu.sync_copy(t_hbm.at[pl.ds(off, C_SCAT)], t_v)
                pltpu.sync_copy(s_hbm.at[pl.ds(off, C_SCAT)], s_v)

                def grp(g, carry2):
                    for u in range(UNROLL):
                        off2 = (g * UNROLL + u) * L
                        hh = h_v[pl.ds(off2, L)]
                        tt = t_v[pl.ds(off2, L)]
                        ss = s_v[pl.ds(off2, L)]
                        rel = hh - base_row
                        m = (rel >= 0) & (rel < ROWS_B)
                        idx = jnp.where(m, rel * N_ENT + tt, 0)
                        plsc.addupdate_scatter(blk_v, [idx], ss, mask=m)
                    return carry2

                lax.fori_loop(0, C_SCAT // (L * UNROLL), grp, 0)
                return carry

            lax.fori_loop(0, N_CHUNKS2, chunk, 0)
            pltpu.sync_copy(blk_v.at[pl.ds(0, BLK_W)],
                            out_hbm.at[pl.ds(blk * BLK_W, BLK_W)])


def _make_scatter_call():
    return functools.partial(
        pl.kernel,
        out_type=jax.ShapeDtypeStruct((N_ENT * N_ENT,), jnp.float32),
        mesh=_mesh(),
        scratch_types=[
            pltpu.VMEM((BLK_WP,), jnp.float32),
            pltpu.VMEM((C_SCAT,), jnp.int32),
            pltpu.VMEM((C_SCAT,), jnp.int32),
            pltpu.VMEM((C_SCAT,), jnp.float32),
        ],
        compiler_params=pltpu.CompilerParams(needs_layout_passes=False),
    )(_scatter_kernel)


def _score_finish_body(sq_ref, out_ref):
    sq = sq_ref[:]
    out_ref[:] = 1.0 - jnp.sqrt(sq) * jnp.float32(1.0 / (3.0 * math.sqrt(D)))


def _score_finish(sq):
    sq2 = sq.reshape(E_PAD // 128, 128)
    out = pl.pallas_call(
        _score_finish_body,
        out_shape=jax.ShapeDtypeStruct((E_PAD // 128, 128), jnp.float32),
    )(sq2)
    return out.reshape(E_PAD)


def _greedy_body(a_ref, b_ref, osr_ref, otg_ref):
    n = a_ref.shape[0]
    a = a_ref[:]
    b = b_ref[:]
    an = a / (jnp.sqrt(jnp.sum(a * a, axis=1, keepdims=True)) + 1e-8)
    bn = b / (jnp.sqrt(jnp.sum(b * b, axis=1, keepdims=True)) + 1e-8)
    sim = lax.dot_general(an, bn, (((1,), (1,)), ((), ())),
                          preferred_element_type=jnp.float32)
    ir = lax.broadcasted_iota(jnp.int32, (n, n), 0)
    ic = lax.broadcasted_iota(jnp.int32, (n, n), 1)
    iota1 = ir[:, 0]

    def it(_, state):
        s, cor, roc = state
        gm = jnp.max(s)
        hit = s == gm
        r = jnp.min(jnp.where(hit, ir, BIG))
        c = jnp.min(jnp.where(hit & (ir == r), ic, BIG))
        cor = jnp.where(iota1 == r, c, cor)
        roc = jnp.where(iota1 == c, r, roc)
        s = jnp.where((ir == r) | (ic == c), NEG, s)
        return (s, cor, roc)

    zi = jnp.zeros((n,), jnp.int32)
    _, cor, roc = lax.fori_loop(0, n, it, (sim, zi, zi))
    r_sr = jnp.sum(jnp.where(ic == cor[:, None], sim, 0.0), axis=1)
    r_tg = jnp.sum(jnp.where(ir == roc[None, :], sim, 0.0), axis=0)
    osr_ref[0, :] = r_sr
    otg_ref[0, :] = r_tg


def _greedy_call(a_pad, b):
    n = b.shape[0]
    return pl.pallas_call(
        _greedy_body,
        out_shape=[
            jax.ShapeDtypeStruct((1, n), jnp.float32),
            jax.ShapeDtypeStruct((1, n), jnp.float32),
        ],
    )(a_pad, b)


def kernel(entity_emb_sr, entity_emb_tg, relation_emb_sr, relation_emb_tg,
           head_sr, tail_sr, relation_sr, head_tg, tail_tg, relation_tg):
    n_sr, n_tg = relation_emb_sr.shape[0], relation_emb_tg.shape[0]
    a_pad = jnp.concatenate(
        [relation_emb_sr, jnp.zeros((n_tg - n_sr, D), jnp.float32)], axis=0)
    r_sr2, r_tg2 = _greedy_call(a_pad, relation_emb_tg)
    r_sim_sr = r_sr2[0, :n_sr]
    r_sim_tg = r_tg2[0, :]

    pad = E_PAD - E_TRI
    zpad = jnp.zeros((pad,), jnp.int32)
    spad = jnp.full((pad,), BIG, jnp.int32)

    score_call = _make_score_call()
    scatter_call = _make_scatter_call()

    def one_graph(ent_emb, rel_emb, head, tail, rel):
        hg = jnp.concatenate([head, zpad])
        tg_ = jnp.concatenate([tail, zpad])
        rg = jnp.concatenate([rel, zpad])
        hs = jnp.concatenate([head, spad])
        sq = score_call(ent_emb, rel_emb, hg, tg_, rg)
        scores = _score_finish(sq)
        m = scatter_call(hs, tg_, scores)
        return m.reshape(N_ENT, N_ENT)

    score_m_sr = one_graph(entity_emb_sr, relation_emb_sr,
                           head_sr, tail_sr, relation_sr)
    score_m_tg = one_graph(entity_emb_tg, relation_emb_tg,
                           head_tg, tail_tg, relation_tg)
    return (r_sim_sr, r_sim_tg, score_m_sr, score_m_tg)


# speedup vs baseline: 1.1855x; 1.1855x over previous
import functools
import math

import jax
import jax.numpy as jnp
from jax import lax
from jax.experimental import pallas as pl
from jax.experimental.pallas import tpu as pltpu
from jax.experimental.pallas import tpu_sc as plsc

N_ENT = 5000
D = 256
E_TRI = 150000

NC = 2
NS = 16
NW = NC * NS
L = 16

C_SCORE = 64
E_PAD = ((E_TRI + NW * C_SCORE - 1) // (NW * C_SCORE)) * (NW * C_SCORE)
PER_W = E_PAD // NW
N_CHUNKS = PER_W // C_SCORE

ROWS_B = 20
N_BLK = N_ENT // ROWS_B
BLK_W = ROWS_B * N_ENT
BLK_WP = ((BLK_W + L - 1) // L) * L
C_SCAT = 2048
UNROLL = 8
N_CHUNKS2 = E_PAD // C_SCAT
N_PASS = (N_BLK + NW - 1) // NW

NEG = -1000000000.0
BIG = 1 << 30


def _mesh():
    return plsc.VectorSubcoreMesh(
        core_axis_name="c", subcore_axis_name="s", num_cores=NC, num_subcores=NS
    )


def _score_kernel(ent_hbm, rel_hbm, h_hbm, t_hbm, r_hbm, out_hbm,
                  hi_v, ti_v, ri_v, hrow_v, trow_v, rrow_v, sc_v, sem):
    wid = lax.axis_index("s") * NC + lax.axis_index("c")
    base_w = wid * PER_W
    lane = lax.iota(jnp.int32, L)

    def chunk(ci, carry):
        base = base_w + ci * C_SCORE
        pltpu.sync_copy(h_hbm.at[pl.ds(base, C_SCORE)], hi_v)
        pltpu.sync_copy(t_hbm.at[pl.ds(base, C_SCORE)], ti_v)
        pltpu.sync_copy(r_hbm.at[pl.ds(base, C_SCORE)], ri_v)
        pltpu.async_copy(ent_hbm.at[hi_v], hrow_v, sem).wait()
        pltpu.async_copy(ent_hbm.at[ti_v], trow_v, sem).wait()
        pltpu.async_copy(rel_hbm.at[ri_v], rrow_v, sem).wait()

        def grp(g, carry2):
            sqv = jnp.zeros((L,), jnp.float32)
            for i in range(L):
                acc = jnp.zeros((L,), jnp.float32)
                row = g * L + i
                for j in range(D // L):
                    hv = hrow_v[row, pl.ds(j * L, L)]
                    rv = rrow_v[row, pl.ds(j * L, L)]
                    tv = trow_v[row, pl.ds(j * L, L)]
                    d = hv + rv - tv
                    acc = acc + d * d
                for k in (8, 4, 2, 1):
                    acc = acc + acc.at[lane ^ k].get(mode="promise_in_bounds")
                sqv = jnp.where(lane == i, acc, sqv)
            sc_v[pl.ds(g * L, L)] = sqv
            return carry2

        lax.fori_loop(0, C_SCORE // L, grp, 0)
        pltpu.sync_copy(sc_v, out_hbm.at[pl.ds(base, C_SCORE)])
        return carry

    lax.fori_loop(0, N_CHUNKS, chunk, 0)


def _make_score_call():
    return functools.partial(
        pl.kernel,
        out_type=jax.ShapeDtypeStruct((E_PAD,), jnp.float32),
        mesh=_mesh(),
        scratch_types=[
            pltpu.VMEM((C_SCORE,), jnp.int32),
            pltpu.VMEM((C_SCORE,), jnp.int32),
            pltpu.VMEM((C_SCORE,), jnp.int32),
            pltpu.VMEM((C_SCORE, D), jnp.float32),
            pltpu.VMEM((C_SCORE, D), jnp.float32),
            pltpu.VMEM((C_SCORE, D), jnp.float32),
            pltpu.VMEM((C_SCORE,), jnp.float32),
            pltpu.SemaphoreType.DMA,
        ],
        compiler_params=pltpu.CompilerParams(needs_layout_passes=False),
    )(_score_kernel)


def _scatter_kernel(h_hbm, t_hbm, s_hbm, out_hbm, blk_v, h_v, t_v, s_v):
    wid = lax.axis_index("s") * NC + lax.axis_index("c")

    for p in range(N_PASS):
        blk = wid + NW * p

        @pl.when(blk < N_BLK)
        def _():
            base_row = blk * ROWS_B

            def zero(i, carry):
                blk_v[pl.ds(i * L, L)] = jnp.zeros((L,), jnp.float32)
                return carry

            lax.fori_loop(0, BLK_WP // L, zero, 0)

            def chunk(ci, carry):
                off = ci * C_SCAT
                pltpu.sync_copy(h_hbm.at[pl.ds(off, C_SCAT)], h_v)
                pltpu.sync_copy(t_hbm.at[pl.ds(off, C_SCAT)], t_v)
                pltpu.sync_copy(s_hbm.at[pl.ds(off, C_SCAT)], s_v)

                def grp(g, carry2):
                    for u in range(UNROLL):
                        off2 = (g * UNROLL + u) * L
                        hh = h_v[pl.ds(off2, L)]
                        tt = t_v[pl.ds(off2, L)]
                        ss = s_v[pl.ds(off2, L)]
                        rel = hh - base_row
                        m = (rel >= 0) & (rel < ROWS_B)
                        idx = jnp.where(m, rel * N_ENT + tt, 0)
                        plsc.addupdate_scatter(blk_v, [idx], ss, mask=m)
                    return carry2

                lax.fori_loop(0, C_SCAT // (L * UNROLL), grp, 0)
                return carry

            lax.fori_loop(0, N_CHUNKS2, chunk, 0)
            pltpu.sync_copy(blk_v.at[pl.ds(0, BLK_W)],
                            out_hbm.at[pl.ds(blk * BLK_W, BLK_W)])


def _make_scatter_call():
    return functools.partial(
        pl.kernel,
        out_type=jax.ShapeDtypeStruct((N_ENT * N_ENT,), jnp.float32),
        mesh=_mesh(),
        scratch_types=[
            pltpu.VMEM((BLK_WP,), jnp.float32),
            pltpu.VMEM((C_SCAT,), jnp.int32),
            pltpu.VMEM((C_SCAT,), jnp.int32),
            pltpu.VMEM((C_SCAT,), jnp.float32),
        ],
        compiler_params=pltpu.CompilerParams(needs_layout_passes=False),
    )(_scatter_kernel)


def _score_finish_body(sq_ref, out_ref):
    sq = sq_ref[:]
    out_ref[:] = 1.0 - jnp.sqrt(sq) * jnp.float32(1.0 / (3.0 * math.sqrt(D)))


def _score_finish(sq):
    sq2 = sq.reshape(E_PAD // 128, 128)
    out = pl.pallas_call(
        _score_finish_body,
        out_shape=jax.ShapeDtypeStruct((E_PAD // 128, 128), jnp.float32),
    )(sq2)
    return out.reshape(E_PAD)


def _greedy_body(a_ref, b_ref, osr_ref, otg_ref):
    n = a_ref.shape[0]
    a = a_ref[:]
    b = b_ref[:]
    an = a / (jnp.sqrt(jnp.sum(a * a, axis=1, keepdims=True)) + 1e-8)
    bn = b / (jnp.sqrt(jnp.sum(b * b, axis=1, keepdims=True)) + 1e-8)
    sim = lax.dot_general(an, bn, (((1,), (1,)), ((), ())),
                          preferred_element_type=jnp.float32)
    ir = lax.broadcasted_iota(jnp.int32, (n, n), 0)
    ic = lax.broadcasted_iota(jnp.int32, (n, n), 1)
    iota1 = ir[:, 0]

    def it(_, state):
        s, cor, roc = state
        gm = jnp.max(s)
        hit = s == gm
        r = jnp.min(jnp.where(hit, ir, BIG))
        c = jnp.min(jnp.where(hit & (ir == r), ic, BIG))
        cor = jnp.where(iota1 == r, c, cor)
        roc = jnp.where(iota1 == c, r, roc)
        s = jnp.where((ir == r) | (ic == c), NEG, s)
        return (s, cor, roc)

    zi = jnp.zeros((n,), jnp.int32)
    _, cor, roc = lax.fori_loop(0, n, it, (sim, zi, zi))
    r_sr = jnp.sum(jnp.where(ic == cor[:, None], sim, 0.0), axis=1)
    r_tg = jnp.sum(jnp.where(ir == roc[None, :], sim, 0.0), axis=0)
    osr_ref[0, :] = r_sr
    otg_ref[0, :] = r_tg


def _greedy_call(a_pad, b):
    n = b.shape[0]
    return pl.pallas_call(
        _greedy_body,
        out_shape=[
            jax.ShapeDtypeStruct((1, n), jnp.float32),
            jax.ShapeDtypeStruct((1, n), jnp.float32),
        ],
    )(a_pad, b)


def kernel(entity_emb_sr, entity_emb_tg, relation_emb_sr, relation_emb_tg,
           head_sr, tail_sr, relation_sr, head_tg, tail_tg, relation_tg):
    n_sr, n_tg = relation_emb_sr.shape[0], relation_emb_tg.shape[0]
    a_pad = jnp.concatenate(
        [relation_emb_sr, jnp.zeros((n_tg - n_sr, D), jnp.float32)], axis=0)
    r_sr2, r_tg2 = _greedy_call(a_pad, relation_emb_tg)
    r_sim_sr = r_sr2[0, :n_sr]
    r_sim_tg = r_tg2[0, :]

    pad = E_PAD - E_TRI
    zpad = jnp.zeros((pad,), jnp.int32)
    spad = jnp.full((pad,), BIG, jnp.int32)

    score_call = _make_score_call()
    scatter_call = _make_scatter_call()

    def one_graph(ent_emb, rel_emb, head, tail, rel):
        hg = jnp.concatenate([head, zpad])
        tg_ = jnp.concatenate([tail, zpad])
        rg = jnp.concatenate([rel, zpad])
        hs = jnp.concatenate([head, spad])
        sq = score_call(ent_emb, rel_emb, hg, tg_, rg)
        scores = _score_finish(sq)
        m = scatter_call(hs, tg_, scores)
        return m.reshape(N_ENT, N_ENT)

    score_m_sr = one_graph(entity_emb_sr, relation_emb_sr,
                           head_sr, tail_sr, relation_sr)
    score_m_tg = one_graph(entity_emb_tg, relation_emb_tg,
                           head_tg, tail_tg, relation_tg)
    return (r_sim_sr, r_sim_tg, score_m_sr, score_m_tg)


# speedup vs baseline: 1.2389x; 1.0450x over previous
import functools
import math

import jax
import jax.numpy as jnp
from jax import lax
from jax.experimental import pallas as pl
from jax.experimental.pallas import tpu as pltpu
from jax.experimental.pallas import tpu_sc as plsc

N_ENT = 5000
D = 256
E_TRI = 150000

NC = 2
NS = 16
NW = NC * NS
L = 16

C_SCORE = 128
E_PAD = ((E_TRI + NW * C_SCORE - 1) // (NW * C_SCORE)) * (NW * C_SCORE)
PER_W = E_PAD // NW
N_CHUNKS = PER_W // C_SCORE

ROWS_B = 20
N_BLK = N_ENT // ROWS_B
BLK_W = ROWS_B * N_ENT
BLK_WP = ((BLK_W + L - 1) // L) * L
C_SCAT = 2048
UNROLL = 8
N_CHUNKS2 = E_PAD // C_SCAT
N_PASS = (N_BLK + NW - 1) // NW

NEG = -1000000000.0
BIG = 1 << 30


def _mesh():
    return plsc.VectorSubcoreMesh(
        core_axis_name="c", subcore_axis_name="s", num_cores=NC, num_subcores=NS
    )


def _score_kernel(ent_hbm, rel_hbm, h_hbm, t_hbm, r_hbm, out_hbm,
                  hi_v, ti_v, ri_v, hrow_v, trow_v, rrow_v, sc_v, sem):
    wid = lax.axis_index("s") * NC + lax.axis_index("c")
    base_w = wid * PER_W
    lane = lax.iota(jnp.int32, L)

    def chunk(ci, carry):
        base = base_w + ci * C_SCORE
        pltpu.sync_copy(h_hbm.at[pl.ds(base, C_SCORE)], hi_v)
        pltpu.sync_copy(t_hbm.at[pl.ds(base, C_SCORE)], ti_v)
        pltpu.sync_copy(r_hbm.at[pl.ds(base, C_SCORE)], ri_v)
        pltpu.async_copy(ent_hbm.at[hi_v], hrow_v, sem).wait()
        pltpu.async_copy(ent_hbm.at[ti_v], trow_v, sem).wait()
        pltpu.async_copy(rel_hbm.at[ri_v], rrow_v, sem).wait()

        def grp(g, carry2):
            sqv = jnp.zeros((L,), jnp.float32)
            for i in range(L):
                acc = jnp.zeros((L,), jnp.float32)
                row = g * L + i
                for j in range(D // L):
                    hv = hrow_v[row, pl.ds(j * L, L)]
                    rv = rrow_v[row, pl.ds(j * L, L)]
                    tv = trow_v[row, pl.ds(j * L, L)]
                    d = hv + rv - tv
                    acc = acc + d * d
                for k in (8, 4, 2, 1):
                    acc = acc + acc.at[lane ^ k].get(mode="promise_in_bounds")
                sqv = jnp.where(lane == i, acc, sqv)
            sc_v[pl.ds(g * L, L)] = sqv
            return carry2

        lax.fori_loop(0, C_SCORE // L, grp, 0)
        pltpu.sync_copy(sc_v, out_hbm.at[pl.ds(base, C_SCORE)])
        return carry

    lax.fori_loop(0, N_CHUNKS, chunk, 0)


def _make_score_call():
    return functools.partial(
        pl.kernel,
        out_type=jax.ShapeDtypeStruct((E_PAD,), jnp.float32),
        mesh=_mesh(),
        scratch_types=[
            pltpu.VMEM((C_SCORE,), jnp.int32),
            pltpu.VMEM((C_SCORE,), jnp.int32),
            pltpu.VMEM((C_SCORE,), jnp.int32),
            pltpu.VMEM((C_SCORE, D), jnp.float32),
            pltpu.VMEM((C_SCORE, D), jnp.float32),
            pltpu.VMEM((C_SCORE, D), jnp.float32),
            pltpu.VMEM((C_SCORE,), jnp.float32),
            pltpu.SemaphoreType.DMA,
        ],
        compiler_params=pltpu.CompilerParams(needs_layout_passes=False),
    )(_score_kernel)


def _scatter_kernel(h_hbm, t_hbm, s_hbm, out_hbm, blk_v, h_v, t_v, s_v):
    wid = lax.axis_index("s") * NC + lax.axis_index("c")

    for p in range(N_PASS):
        blk = wid + NW * p

        @pl.when(blk < N_BLK)
        def _():
            base_row = blk * ROWS_B

            def zero(i, carry):
                blk_v[pl.ds(i * L, L)] = jnp.zeros((L,), jnp.float32)
                return carry

            lax.fori_loop(0, BLK_WP // L, zero, 0)

            def chunk(ci, carry):
                off = ci * C_SCAT
                pltpu.sync_copy(h_hbm.at[pl.ds(off, C_SCAT)], h_v)
                pltpu.sync_copy(t_hbm.at[pl.ds(off, C_SCAT)], t_v)
                pltpu.sync_copy(s_hbm.at[pl.ds(off, C_SCAT)], s_v)

                def grp(g, carry2):
                    for u in range(UNROLL):
                        off2 = (g * UNROLL + u) * L
                        hh = h_v[pl.ds(off2, L)]
                        tt = t_v[pl.ds(off2, L)]
                        ss = s_v[pl.ds(off2, L)]
                        rel = hh - base_row
                        m = (rel >= 0) & (rel < ROWS_B)
                        idx = jnp.where(m, rel * N_ENT + tt, 0)
                        plsc.addupdate_scatter(blk_v, [idx], ss, mask=m)
                    return carry2

                lax.fori_loop(0, C_SCAT // (L * UNROLL), grp, 0)
                return carry

            lax.fori_loop(0, N_CHUNKS2, chunk, 0)
            pltpu.sync_copy(blk_v.at[pl.ds(0, BLK_W)],
                            out_hbm.at[pl.ds(blk * BLK_W, BLK_W)])


def _make_scatter_call():
    return functools.partial(
        pl.kernel,
        out_type=jax.ShapeDtypeStruct((N_ENT * N_ENT,), jnp.float32),
        mesh=_mesh(),
        scratch_types=[
            pltpu.VMEM((BLK_WP,), jnp.float32),
            pltpu.VMEM((C_SCAT,), jnp.int32),
            pltpu.VMEM((C_SCAT,), jnp.int32),
            pltpu.VMEM((C_SCAT,), jnp.float32),
        ],
        compiler_params=pltpu.CompilerParams(needs_layout_passes=False),
    )(_scatter_kernel)


def _score_finish_body(sq_ref, out_ref):
    sq = sq_ref[:]
    out_ref[:] = 1.0 - jnp.sqrt(sq) * jnp.float32(1.0 / (3.0 * math.sqrt(D)))


def _score_finish(sq):
    sq2 = sq.reshape(E_PAD // 128, 128)
    out = pl.pallas_call(
        _score_finish_body,
        out_shape=jax.ShapeDtypeStruct((E_PAD // 128, 128), jnp.float32),
    )(sq2)
    return out.reshape(E_PAD)


def _greedy_body(a_ref, b_ref, osr_ref, otg_ref):
    n = a_ref.shape[0]
    a = a_ref[:]
    b = b_ref[:]
    an = a / (jnp.sqrt(jnp.sum(a * a, axis=1, keepdims=True)) + 1e-8)
    bn = b / (jnp.sqrt(jnp.sum(b * b, axis=1, keepdims=True)) + 1e-8)
    sim = lax.dot_general(an, bn, (((1,), (1,)), ((), ())),
                          preferred_element_type=jnp.float32)
    ir = lax.broadcasted_iota(jnp.int32, (n, n), 0)
    ic = lax.broadcasted_iota(jnp.int32, (n, n), 1)
    iota1 = ir[:, 0]

    def it(_, state):
        s, cor, roc = state
        gm = jnp.max(s)
        hit = s == gm
        r = jnp.min(jnp.where(hit, ir, BIG))
        c = jnp.min(jnp.where(hit & (ir == r), ic, BIG))
        cor = jnp.where(iota1 == r, c, cor)
        roc = jnp.where(iota1 == c, r, roc)
        s = jnp.where((ir == r) | (ic == c), NEG, s)
        return (s, cor, roc)

    zi = jnp.zeros((n,), jnp.int32)
    _, cor, roc = lax.fori_loop(0, n, it, (sim, zi, zi))
    r_sr = jnp.sum(jnp.where(ic == cor[:, None], sim, 0.0), axis=1)
    r_tg = jnp.sum(jnp.where(ir == roc[None, :], sim, 0.0), axis=0)
    osr_ref[0, :] = r_sr
    otg_ref[0, :] = r_tg


def _greedy_call(a_pad, b):
    n = b.shape[0]
    return pl.pallas_call(
        _greedy_body,
        out_shape=[
            jax.ShapeDtypeStruct((1, n), jnp.float32),
            jax.ShapeDtypeStruct((1, n), jnp.float32),
        ],
    )(a_pad, b)


def kernel(entity_emb_sr, entity_emb_tg, relation_emb_sr, relation_emb_tg,
           head_sr, tail_sr, relation_sr, head_tg, tail_tg, relation_tg):
    n_sr, n_tg = relation_emb_sr.shape[0], relation_emb_tg.shape[0]
    a_pad = jnp.concatenate(
        [relation_emb_sr, jnp.zeros((n_tg - n_sr, D), jnp.float32)], axis=0)
    r_sr2, r_tg2 = _greedy_call(a_pad, relation_emb_tg)
    r_sim_sr = r_sr2[0, :n_sr]
    r_sim_tg = r_tg2[0, :]

    pad = E_PAD - E_TRI
    zpad = jnp.zeros((pad,), jnp.int32)
    spad = jnp.full((pad,), BIG, jnp.int32)

    score_call = _make_score_call()
    scatter_call = _make_scatter_call()

    def one_graph(ent_emb, rel_emb, head, tail, rel):
        hg = jnp.concatenate([head, zpad])
        tg_ = jnp.concatenate([tail, zpad])
        rg = jnp.concatenate([rel, zpad])
        hs = jnp.concatenate([head, spad])
        sq = score_call(ent_emb, rel_emb, hg, tg_, rg)
        scores = _score_finish(sq)
        m = scatter_call(hs, tg_, scores)
        return m.reshape(N_ENT, N_ENT)

    score_m_sr = one_graph(entity_emb_sr, relation_emb_sr,
                           head_sr, tail_sr, relation_sr)
    score_m_tg = one_graph(entity_emb_tg, relation_emb_tg,
                           head_tg, tail_tg, relation_tg)
    return (r_sim_sr, r_sim_tg, score_m_sr, score_m_tg)
